# baseline (speedup 1.0000x reference)
import jax
import jax.numpy as jnp
import numpy as np
from jax.experimental import pallas as pl

B = 32
H = 16
KV_LEN = 4096
KV_LORA_RANK = 512
QK_ROPE_HEAD_DIM = 64
V_HEAD_DIM = 128
D_MODEL = 4096
D_LAT = KV_LORA_RANK + QK_ROPE_HEAD_DIM
SCALE = 1.0 / np.sqrt(128.0 + 64.0)

N_COL_CHUNKS = 4
COL_CHUNK = D_MODEL // N_COL_CHUNKS


def _attn_kernel(q_ref, kv_ref, o_lat_ref):
    q = q_ref[0].astype(jnp.bfloat16)
    kv = kv_ref[0].astype(jnp.bfloat16)

    s = jax.lax.dot_general(
        q, kv, (((1,), (1,)), ((), ())),
        preferred_element_type=jnp.float32,
    ) * SCALE
    m = jnp.max(s, axis=-1, keepdims=True)
    p = jnp.exp(s - m)
    denom = jnp.sum(p, axis=-1, keepdims=True)

    o_lat_ref[0] = jax.lax.dot_general(
        p.astype(jnp.bfloat16), kv[:, :KV_LORA_RANK], (((1,), (0,)), ((), ())),
        preferred_element_type=jnp.float32,
    ) / denom


def _proj_kernel(o_lat_ref, w_uv_ref, w_o_ref, out_ref):
    o = jax.lax.dot_general(
        o_lat_ref[...].astype(jnp.bfloat16),
        w_uv_ref[...].astype(jnp.bfloat16),
        (((2,), (1,)), ((1,), (0,))),
        preferred_element_type=jnp.float32,
    )
    o = o.transpose(1, 0, 2).reshape(B, H * V_HEAD_DIM)
    out_ref[...] = jax.lax.dot_general(
        o.astype(jnp.bfloat16), w_o_ref[...].astype(jnp.bfloat16),
        (((1,), (0,)), ((), ())),
        preferred_element_type=jnp.float32,
    )


@jax.jit
def kernel(q_nope, q_pe, kv_cache, w_uv, w_o):
    q = jnp.concatenate([q_nope, q_pe], axis=-1)

    o_lat = pl.pallas_call(
        _attn_kernel,
        grid=(B,),
        in_specs=[
            pl.BlockSpec((1, H, D_LAT), lambda b: (b, 0, 0)),
            pl.BlockSpec((1, KV_LEN, D_LAT), lambda b: (b, 0, 0)),
        ],
        out_specs=pl.BlockSpec((1, H, KV_LORA_RANK), lambda b: (b, 0, 0)),
        out_shape=jax.ShapeDtypeStruct((B, H, KV_LORA_RANK), jnp.float32),
    )(q, kv_cache)

    out = pl.pallas_call(
        _proj_kernel,
        grid=(N_COL_CHUNKS,),
        in_specs=[
            pl.BlockSpec((B, H, KV_LORA_RANK), lambda c: (0, 0, 0)),
            pl.BlockSpec((H, KV_LORA_RANK, V_HEAD_DIM), lambda c: (0, 0, 0)),
            pl.BlockSpec((H * V_HEAD_DIM, COL_CHUNK), lambda c: (0, c)),
        ],
        out_specs=pl.BlockSpec((B, COL_CHUNK), lambda c: (0, c)),
        out_shape=jax.ShapeDtypeStruct((B, D_MODEL), jnp.float32),
    )(o_lat, w_uv, w_o)
    return out


# speedup vs baseline: 1.0024x; 1.0024x over previous
import jax
import jax.numpy as jnp
import numpy as np
from jax.experimental import pallas as pl

B = 32
H = 16
KV_LEN = 4096
KV_LORA_RANK = 512
QK_ROPE_HEAD_DIM = 64
V_HEAD_DIM = 128
D_MODEL = 4096
D_LAT = KV_LORA_RANK + QK_ROPE_HEAD_DIM
SCALE = 1.0 / np.sqrt(128.0 + 64.0)

N_KV_SPLIT = 4
KV_CHUNK = KV_LEN // N_KV_SPLIT

N_COL_CHUNKS = 4
COL_CHUNK = D_MODEL // N_COL_CHUNKS
N_ROW_SPLIT = 2
ROW_CHUNK = (H * V_HEAD_DIM) // N_ROW_SPLIT


def _attn_kernel(q_ref, *kv_refs_and_out):
    kv_refs = kv_refs_and_out[:N_KV_SPLIT]
    o_lat_ref = kv_refs_and_out[N_KV_SPLIT]
    q = q_ref[0].astype(jnp.bfloat16)

    kvs = [r[0].astype(jnp.bfloat16) for r in kv_refs]
    s = jnp.concatenate(
        [
            jax.lax.dot_general(
                q, kv, (((1,), (1,)), ((), ())),
                preferred_element_type=jnp.float32,
            )
            for kv in kvs
        ],
        axis=1,
    ) * SCALE
    m = jnp.max(s, axis=-1, keepdims=True)
    p_f32 = jnp.exp(s - m)
    p = p_f32.astype(jnp.bfloat16)
    denom = jnp.sum(p_f32, axis=-1, keepdims=True)

    acc = jnp.zeros((H, KV_LORA_RANK), jnp.float32)
    for i, kv in enumerate(kvs):
        acc = acc + jax.lax.dot_general(
            p[:, i * KV_CHUNK:(i + 1) * KV_CHUNK], kv[:, :KV_LORA_RANK],
            (((1,), (0,)), ((), ())),
            preferred_element_type=jnp.float32,
        )
    o_lat_ref[0] = acc / denom


def _proj_kernel(o_lat_ref, w_uv_ref, w_o_a_ref, w_o_b_ref, out_ref):
    o = jax.lax.dot_general(
        o_lat_ref[...].astype(jnp.bfloat16),
        w_uv_ref[...].astype(jnp.bfloat16),
        (((2,), (1,)), ((1,), (0,))),
        preferred_element_type=jnp.float32,
    )
    o = o.transpose(1, 0, 2).reshape(B, H * V_HEAD_DIM).astype(jnp.bfloat16)
    out_ref[...] = jax.lax.dot_general(
        o[:, :ROW_CHUNK], w_o_a_ref[...].astype(jnp.bfloat16),
        (((1,), (0,)), ((), ())),
        preferred_element_type=jnp.float32,
    ) + jax.lax.dot_general(
        o[:, ROW_CHUNK:], w_o_b_ref[...].astype(jnp.bfloat16),
        (((1,), (0,)), ((), ())),
        preferred_element_type=jnp.float32,
    )


@jax.jit
def kernel(q_nope, q_pe, kv_cache, w_uv, w_o):
    q = jnp.concatenate([q_nope, q_pe], axis=-1)

    kv_specs = [
        pl.BlockSpec(
            (1, KV_CHUNK, D_LAT),
            lambda b, i=i: (b, i, 0),
        )
        for i in range(N_KV_SPLIT)
    ]
    o_lat = pl.pallas_call(
        _attn_kernel,
        grid=(B,),
        in_specs=[pl.BlockSpec((1, H, D_LAT), lambda b: (b, 0, 0))] + kv_specs,
        out_specs=pl.BlockSpec((1, H, KV_LORA_RANK), lambda b: (b, 0, 0)),
        out_shape=jax.ShapeDtypeStruct((B, H, KV_LORA_RANK), jnp.float32),
    )(q, *([kv_cache] * N_KV_SPLIT))

    out = pl.pallas_call(
        _proj_kernel,
        grid=(N_COL_CHUNKS,),
        in_specs=[
            pl.BlockSpec((B, H, KV_LORA_RANK), lambda c: (0, 0, 0)),
            pl.BlockSpec((H, KV_LORA_RANK, V_HEAD_DIM), lambda c: (0, 0, 0)),
            pl.BlockSpec((ROW_CHUNK, COL_CHUNK), lambda c: (0, c)),
            pl.BlockSpec((ROW_CHUNK, COL_CHUNK), lambda c: (1, c)),
        ],
        out_specs=pl.BlockSpec((B, COL_CHUNK), lambda c: (0, c)),
        out_shape=jax.ShapeDtypeStruct((B, D_MODEL), jnp.float32),
    )(o_lat, w_uv, w_o, w_o)
    return out


# speedup vs baseline: 3.6030x; 3.5943x over previous
import jax
import jax.numpy as jnp
import numpy as np
from jax.experimental import pallas as pl

B = 32
H = 16
KV_LEN = 4096
KV_LORA_RANK = 512
QK_ROPE_HEAD_DIM = 64
V_HEAD_DIM = 128
D_MODEL = 4096
D_LAT = KV_LORA_RANK + QK_ROPE_HEAD_DIM
SCALE = 1.0 / np.sqrt(128.0 + 64.0)

N_COL_CHUNKS = 4
COL_CHUNK = D_MODEL // N_COL_CHUNKS


def _attn_kernel(q_ref, kvt_ref, o_lat_ref):
    q = q_ref[0].astype(jnp.bfloat16)
    kvt = kvt_ref[0].astype(jnp.bfloat16)

    s = jax.lax.dot_general(
        q, kvt, (((1,), (0,)), ((), ())),
        preferred_element_type=jnp.float32,
    ) * SCALE
    m = jnp.max(s, axis=-1, keepdims=True)
    p_f32 = jnp.exp(s - m)
    p = p_f32.astype(jnp.bfloat16)
    denom = jnp.sum(p_f32, axis=-1, keepdims=True)

    o_lat_ref[0] = jax.lax.dot_general(
        p, kvt[:KV_LORA_RANK, :], (((1,), (1,)), ((), ())),
        preferred_element_type=jnp.float32,
    ) / denom


def _proj_kernel(o_lat_ref, w_uv_ref, w_o_ref, out_ref):
    o = jax.lax.dot_general(
        o_lat_ref[...].astype(jnp.bfloat16),
        w_uv_ref[...].astype(jnp.bfloat16),
        (((2,), (1,)), ((1,), (0,))),
        preferred_element_type=jnp.float32,
    )
    o = o.transpose(1, 0, 2).reshape(B, H * V_HEAD_DIM).astype(jnp.bfloat16)
    out_ref[...] = jax.lax.dot_general(
        o, w_o_ref[...].astype(jnp.bfloat16),
        (((1,), (0,)), ((), ())),
        preferred_element_type=jnp.float32,
    )


@jax.jit
def kernel(q_nope, q_pe, kv_cache, w_uv, w_o):
    q = jnp.concatenate([q_nope, q_pe], axis=-1)
    kv_t = jnp.transpose(kv_cache, (0, 2, 1))

    o_lat = pl.pallas_call(
        _attn_kernel,
        grid=(B,),
        in_specs=[
            pl.BlockSpec((1, H, D_LAT), lambda b: (b, 0, 0)),
            pl.BlockSpec((1, D_LAT, KV_LEN), lambda b: (b, 0, 0)),
        ],
        out_specs=pl.BlockSpec((1, H, KV_LORA_RANK), lambda b: (b, 0, 0)),
        out_shape=jax.ShapeDtypeStruct((B, H, KV_LORA_RANK), jnp.float32),
    )(q, kv_t)

    out = pl.pallas_call(
        _proj_kernel,
        grid=(N_COL_CHUNKS,),
        in_specs=[
            pl.BlockSpec((B, H, KV_LORA_RANK), lambda c: (0, 0, 0)),
            pl.BlockSpec((H, KV_LORA_RANK, V_HEAD_DIM), lambda c: (0, 0, 0)),
            pl.BlockSpec((H * V_HEAD_DIM, COL_CHUNK), lambda c: (0, c)),
        ],
        out_specs=pl.BlockSpec((B, COL_CHUNK), lambda c: (0, c)),
        out_shape=jax.ShapeDtypeStruct((B, D_MODEL), jnp.float32),
    )(o_lat, w_uv, w_o)
    return out


# speedup vs baseline: 3.8190x; 1.0599x over previous
import jax
import jax.numpy as jnp
import numpy as np
from jax.experimental import pallas as pl
from jax.experimental.pallas import tpu as pltpu

B = 32
H = 16
KV_LEN = 4096
KV_LORA_RANK = 512
QK_ROPE_HEAD_DIM = 64
V_HEAD_DIM = 128
D_MODEL = 4096
D_LAT = KV_LORA_RANK + QK_ROPE_HEAD_DIM
SCALE = 1.0 / np.sqrt(128.0 + 64.0)

B_BLK = 2
N_COL_CHUNKS = 4
COL_CHUNK = D_MODEL // N_COL_CHUNKS


def _attn_kernel(qn_ref, qp_ref, kvt_ref, o_lat_ref):
    qn = qn_ref[...].astype(jnp.bfloat16)
    qp = qp_ref[...].astype(jnp.bfloat16)
    kvt = kvt_ref[...].astype(jnp.bfloat16)

    lat = kvt[:, :KV_LORA_RANK, :]
    s = jax.lax.dot_general(
        qn, lat, (((2,), (1,)), ((0,), (0,))),
        preferred_element_type=jnp.float32,
    ) + jax.lax.dot_general(
        qp, kvt[:, KV_LORA_RANK:, :], (((2,), (1,)), ((0,), (0,))),
        preferred_element_type=jnp.float32,
    )
    s = s * SCALE
    m = jnp.max(s, axis=-1, keepdims=True)
    p_f32 = jnp.exp(s - m)
    p = p_f32.astype(jnp.bfloat16)
    denom = jnp.sum(p_f32, axis=-1, keepdims=True)

    o_lat_ref[...] = jax.lax.dot_general(
        p, lat, (((2,), (2,)), ((0,), (0,))),
        preferred_element_type=jnp.float32,
    ) / denom


def _proj_kernel(o_lat_ref, w_uv_ref, w_o_ref, out_ref):
    o = jax.lax.dot_general(
        o_lat_ref[...].astype(jnp.bfloat16),
        w_uv_ref[...].astype(jnp.bfloat16),
        (((2,), (1,)), ((1,), (0,))),
        preferred_element_type=jnp.float32,
    )
    o = o.transpose(1, 0, 2).reshape(B, H * V_HEAD_DIM).astype(jnp.bfloat16)
    out_ref[...] = jax.lax.dot_general(
        o, w_o_ref[...].astype(jnp.bfloat16),
        (((1,), (0,)), ((), ())),
        preferred_element_type=jnp.float32,
    )


@jax.jit
def kernel(q_nope, q_pe, kv_cache, w_uv, w_o):
    kv_t = jnp.transpose(kv_cache, (0, 2, 1))

    o_lat = pl.pallas_call(
        _attn_kernel,
        grid=(B // B_BLK,),
        in_specs=[
            pl.BlockSpec((B_BLK, H, KV_LORA_RANK), lambda b: (b, 0, 0)),
            pl.BlockSpec((B_BLK, H, QK_ROPE_HEAD_DIM), lambda b: (b, 0, 0)),
            pl.BlockSpec((B_BLK, D_LAT, KV_LEN), lambda b: (b, 0, 0)),
        ],
        out_specs=pl.BlockSpec((B_BLK, H, KV_LORA_RANK), lambda b: (b, 0, 0)),
        out_shape=jax.ShapeDtypeStruct((B, H, KV_LORA_RANK), jnp.float32),
        compiler_params=pltpu.CompilerParams(
            dimension_semantics=("arbitrary",),
        ),
    )(q_nope, q_pe, kv_t)

    out = pl.pallas_call(
        _proj_kernel,
        grid=(N_COL_CHUNKS,),
        in_specs=[
            pl.BlockSpec((B, H, KV_LORA_RANK), lambda c: (0, 0, 0)),
            pl.BlockSpec((H, KV_LORA_RANK, V_HEAD_DIM), lambda c: (0, 0, 0)),
            pl.BlockSpec((H * V_HEAD_DIM, COL_CHUNK), lambda c: (0, c)),
        ],
        out_specs=pl.BlockSpec((B, COL_CHUNK), lambda c: (0, c)),
        out_shape=jax.ShapeDtypeStruct((B, D_MODEL), jnp.float32),
        compiler_params=pltpu.CompilerParams(
            dimension_semantics=("arbitrary",),
        ),
    )(o_lat, w_uv, w_o)
    return out
